# baseline (speedup 1.0000x reference)
import functools

import jax
import jax.numpy as jnp
from jax import lax
from jax.experimental import pallas as pl
from jax.experimental.pallas import tpu as pltpu
from jax.experimental.pallas import tpu_sc as plsc

B = 16384
L = 16
TOP = 1111111
M = 1000000

_info = plsc.get_sparse_core_info()
NC = _info.num_cores
NS = _info.num_subcores
NW = NC * NS
BPW = B // NW


def _sc_body(n_hbm, o_hbm, out_hbm, n_v, o_v, out_v):
    wid = lax.axis_index("s") * NC + lax.axis_index("c")
    start = wid * BPW
    pltpu.sync_copy(n_hbm.at[pl.ds(start, BPW)], n_v)
    pltpu.sync_copy(o_hbm.at[pl.ds(start, BPW)], o_v)
    for i in range(BPW // L):
        nv = n_v[pl.ds(i * L, L)]
        ov = o_v[pl.ds(i * L, L)]
        r = nv - TOP
        for t in (8 * M, 4 * M, 2 * M, M):
            r = jnp.where(r >= t, r - t, r)
        lo = nv * 10 + (ov + 1)
        hi = r * 10 + (ov + TOP)
        out_v[pl.ds(i * L, L)] = jnp.where(nv >= TOP, hi, lo)
    pltpu.sync_copy(out_v, out_hbm.at[pl.ds(start, BPW)])


_sc_call = functools.partial(
    pl.kernel,
    mesh=plsc.VectorSubcoreMesh(core_axis_name="c", subcore_axis_name="s"),
    out_type=jax.ShapeDtypeStruct((B,), jnp.int32),
    scratch_types=[
        pltpu.VMEM((BPW,), jnp.int32),
        pltpu.VMEM((BPW,), jnp.int32),
        pltpu.VMEM((BPW,), jnp.int32),
    ],
)(_sc_body)


def kernel(n, o):
    n1 = _sc_call(n, o)
    return (n1, jnp.zeros((), dtype=n1.dtype))


# speedup vs baseline: 1.0558x; 1.0558x over previous
import functools

import jax
import jax.numpy as jnp
from jax import lax
from jax.experimental import pallas as pl
from jax.experimental.pallas import tpu as pltpu
from jax.experimental.pallas import tpu_sc as plsc

B = 16384
L = 16
TOP = 1111111
M = 1000000

_info = plsc.get_sparse_core_info()
NC = 1
NS = _info.num_subcores
NW = NC * NS
BPW = B // NW


def _sc_body(n_hbm, o_hbm, out_hbm, n_v, o_v, out_v, sem_n, sem_o):
    wid = lax.axis_index("s") * NC + lax.axis_index("c")
    start = wid * BPW
    cp_n = pltpu.async_copy(n_hbm.at[pl.ds(start, BPW)], n_v, sem_n)
    cp_o = pltpu.async_copy(o_hbm.at[pl.ds(start, BPW)], o_v, sem_o)
    cp_n.wait()
    cp_o.wait()
    for i in range(BPW // L):
        nv = n_v[pl.ds(i * L, L)]
        ov = o_v[pl.ds(i * L, L)]
        r = nv - TOP
        for t in (8 * M, 4 * M, 2 * M, M):
            r = jnp.where(r >= t, r - t, r)
        lo = nv * 10 + (ov + 1)
        hi = r * 10 + (ov + TOP)
        out_v[pl.ds(i * L, L)] = jnp.where(nv >= TOP, hi, lo)
    pltpu.sync_copy(out_v, out_hbm.at[pl.ds(start, BPW)])


_sc_call = functools.partial(
    pl.kernel,
    mesh=plsc.VectorSubcoreMesh(
        core_axis_name="c", subcore_axis_name="s", num_cores=NC),
    out_type=jax.ShapeDtypeStruct((B,), jnp.int32),
    scratch_types=[
        pltpu.VMEM((BPW,), jnp.int32),
        pltpu.VMEM((BPW,), jnp.int32),
        pltpu.VMEM((BPW,), jnp.int32),
        pltpu.SemaphoreType.DMA,
        pltpu.SemaphoreType.DMA,
    ],
)(_sc_body)


def kernel(n, o):
    n1 = _sc_call(n, o)
    return (n1, jnp.zeros((), dtype=n1.dtype))


# speedup vs baseline: 1.0576x; 1.0018x over previous
import functools

import jax
import jax.numpy as jnp
from jax import lax
from jax.experimental import pallas as pl
from jax.experimental.pallas import tpu as pltpu
from jax.experimental.pallas import tpu_sc as plsc

B = 16384
L = 16
TOP = 1111111
M = 1000000

_info = plsc.get_sparse_core_info()
NC = 1
NS = _info.num_subcores
NW = NC * NS
BPW = B // NW


def _sc_body(n_hbm, o_hbm, out_hbm, n_v, o_v, out_v, sem_n, sem_o):
    wid = lax.axis_index("s") * NC + lax.axis_index("c")
    start = wid * BPW
    cp_n = pltpu.async_copy(n_hbm.at[pl.ds(start, BPW)], n_v, sem_n)
    cp_o = pltpu.async_copy(o_hbm.at[pl.ds(start, BPW)], o_v, sem_o)
    cp_n.wait()
    cp_o.wait()
    for i in range(BPW // L):
        nv = n_v[pl.ds(i * L, L)]
        ov = o_v[pl.ds(i * L, L)]
        r = nv - TOP
        for t in (8 * M, 4 * M, 2 * M, M):
            r = jnp.where(r >= t, r - t, r)
        cond = nv >= TOP
        m = jnp.where(cond, r, nv)
        c = jnp.where(cond, TOP, 1)
        out_v[pl.ds(i * L, L)] = m * 10 + ov + c
    pltpu.sync_copy(out_v, out_hbm.at[pl.ds(start, BPW)])


_sc_call = functools.partial(
    pl.kernel,
    mesh=plsc.VectorSubcoreMesh(
        core_axis_name="c", subcore_axis_name="s", num_cores=NC),
    out_type=jax.ShapeDtypeStruct((B,), jnp.int32),
    scratch_types=[
        pltpu.VMEM((BPW,), jnp.int32),
        pltpu.VMEM((BPW,), jnp.int32),
        pltpu.VMEM((BPW,), jnp.int32),
        pltpu.SemaphoreType.DMA,
        pltpu.SemaphoreType.DMA,
    ],
)(_sc_body)


def kernel(n, o):
    n1 = _sc_call(n, o)
    return (n1, jnp.zeros((), dtype=n1.dtype))
